# Initial kernel scaffold; baseline (speedup 1.0000x reference)
import jax
import jax.numpy as jnp
from jax.experimental import pallas as pl


def kernel(x, edge_index, Wl0, bl0, Wr0, Wl1, bl1, Wr1, Wl2, bl2, Wr2, Wl3, bl3, Wr3):
    raise NotImplementedError("write your pallas kernel here")


# speedup vs baseline: 4.3990x; 4.3990x over previous
import functools

import jax
import jax.numpy as jnp
from jax import lax
from jax.experimental import pallas as pl
from jax.experimental.pallas import tpu as pltpu
from jax.experimental.pallas import tpu_sc as plsc

N = 10000
E = 320000
NCORES = 2
NSUB = 16
NTILES = NCORES * NSUB
CH = 128
EPT = 10240
E_PAD = NTILES * EPT
NCH = EPT // CH
N_ACC = 10240
RPT = N_ACC // NSUB
DPS = (96, 64, 32, 16)
F32 = jnp.float32
_HI = lax.Precision.HIGHEST


def _sc_segsum(dp, with_counts):
    mesh = plsc.VectorSubcoreMesh(core_axis_name="c", subcore_axis_name="s")
    out_type = [jax.ShapeDtypeStruct((NCORES, N_ACC, dp), F32)]
    scratch = [
        pltpu.VMEM((CH,), jnp.int32),
        pltpu.VMEM((CH,), jnp.int32),
        pltpu.VMEM((CH, dp), F32),
        pltpu.VMEM_SHARED((N_ACC, dp), F32),
    ]
    if with_counts:
        out_type.append(jax.ShapeDtypeStruct((NCORES, N_ACC, 16), F32))
        scratch += [
            pltpu.VMEM((CH, 16), F32),
            pltpu.VMEM_SHARED((N_ACC, 16), F32),
        ]

    def body(*refs):
        if with_counts:
            (z_hbm, src_hbm, dst_hbm, zc_hbm, zc16_hbm, ones_hbm,
             out_hbm, cnt_hbm, src_v, dst_v, rows_v, acc_sh, ones_v, cacc_sh) = refs
        else:
            (z_hbm, src_hbm, dst_hbm, zc_hbm,
             out_hbm, src_v, dst_v, rows_v, acc_sh) = refs
        c = lax.axis_index("c")
        s = lax.axis_index("s")
        wid = c * NSUB + s
        lo = s * RPT
        pltpu.sync_copy(zc_hbm.at[pl.ds(lo, RPT)], acc_sh.at[pl.ds(lo, RPT)])
        if with_counts:
            pltpu.sync_copy(zc16_hbm.at[pl.ds(lo, RPT)], cacc_sh.at[pl.ds(lo, RPT)])
            pltpu.sync_copy(ones_hbm, ones_v)
        plsc.subcore_barrier()

        base = wid * EPT

        @pl.loop(0, NCH)
        def _(j):
            off = base + j * CH
            pltpu.sync_copy(src_hbm.at[pl.ds(off, CH)], src_v)
            pltpu.sync_copy(dst_hbm.at[pl.ds(off, CH)], dst_v)
            pltpu.sync_copy(z_hbm.at[src_v], rows_v)
            pltpu.sync_copy(rows_v, acc_sh.at[dst_v], add=True)
            if with_counts:
                pltpu.sync_copy(ones_v, cacc_sh.at[dst_v], add=True)

        plsc.subcore_barrier()
        pltpu.sync_copy(acc_sh.at[pl.ds(lo, RPT)], out_hbm.at[c, pl.ds(lo, RPT)])
        if with_counts:
            pltpu.sync_copy(cacc_sh.at[pl.ds(lo, RPT)], cnt_hbm.at[c, pl.ds(lo, RPT)])

    return pl.kernel(
        body,
        out_type=tuple(out_type) if with_counts else out_type[0],
        mesh=mesh,
        scratch_types=scratch,
        compiler_params=pltpu.CompilerParams(use_tc_tiling_on_sc=False),
    )


def _first_body(x_ref, w_ref, zo_ref):
    zo_ref[...] = jnp.dot(x_ref[...], w_ref[...],
                          preferred_element_type=F32, precision=_HI)


def _tc_first(x, wlt):
    return pl.pallas_call(
        _first_body,
        out_shape=jax.ShapeDtypeStruct((N, wlt.shape[1]), F32),
    )(x, wlt)


def _combine_body(s_ref, c_ref, h_ref, wr_ref, b_ref, wl_ref, ho_ref, zo_ref):
    cnt = c_ref[0, :N, :] + c_ref[1, :N, :]
    inv = 1.0 / jnp.maximum(cnt[:, 0:1], 1.0)
    ssum = s_ref[0, :N, :] + s_ref[1, :N, :]
    hn = ssum * inv + b_ref[...] + jnp.dot(h_ref[...], wr_ref[...],
                                           preferred_element_type=F32, precision=_HI)
    ho_ref[...] = hn
    zo_ref[...] = jnp.dot(hn, wl_ref[...], preferred_element_type=F32, precision=_HI)


def _tc_combine(s, cnt, h, wrt, blp, wlt_next):
    return pl.pallas_call(
        _combine_body,
        out_shape=(jax.ShapeDtypeStruct((N, wrt.shape[1]), F32),
                   jax.ShapeDtypeStruct((N, wlt_next.shape[1]), F32)),
    )(s, cnt, h, wrt, blp, wlt_next)


def _final_body(s_ref, c_ref, h_ref, wr_ref, b_ref, o_ref):
    cnt = c_ref[0, :N, :] + c_ref[1, :N, :]
    inv = 1.0 / jnp.maximum(cnt[:, 0:1], 1.0)
    ssum = s_ref[0, :N, :] + s_ref[1, :N, :]
    o_ref[...] = ssum * inv + b_ref[...] + jnp.dot(h_ref[...], wr_ref[...],
                                                   preferred_element_type=F32, precision=_HI)


def _tc_final(s, cnt, h, wrt, blp):
    return pl.pallas_call(
        _final_body,
        out_shape=jax.ShapeDtypeStruct((N, wrt.shape[1]), F32),
    )(s, cnt, h, wrt, blp)


def _pad_wt(w, dpi, dpo):
    return jnp.zeros((dpi, dpo), F32).at[:w.shape[1], :w.shape[0]].set(w.T)


def _pad_b(b, dpo):
    return jnp.zeros((1, dpo), F32).at[0, :b.shape[0]].set(b)


def kernel(x, edge_index, Wl0, bl0, Wr0, Wl1, bl1, Wr1, Wl2, bl2, Wr2,
           Wl3, bl3, Wr3):
    ei = edge_index.astype(jnp.int32)
    src = jnp.concatenate([ei[0], jnp.zeros((E_PAD - E,), jnp.int32)])
    dst = jnp.concatenate([ei[1], jnp.full((E_PAD - E,), N, jnp.int32)])

    dpi = (128,) + DPS[:-1]
    wls, wrs, bls = (Wl0, Wl1, Wl2, Wl3), (Wr0, Wr1, Wr2, Wr3), (bl0, bl1, bl2, bl3)
    wlt = [_pad_wt(w, i, o) for w, i, o in zip(wls, dpi, DPS)]
    wrt = [_pad_wt(w, i, o) for w, i, o in zip(wrs, dpi, DPS)]
    blp = [_pad_b(b, o) for b, o in zip(bls, DPS)]
    zc = {dp: jnp.zeros((N_ACC, dp), F32) for dp in set(DPS)}
    ones = jnp.ones((CH, 16), F32)

    z0 = _tc_first(x, wlt[0])
    s0, cnt = _sc_segsum(DPS[0], True)(z0, src, dst, zc[DPS[0]], zc[16], ones)
    h1, z1 = _tc_combine(s0, cnt, x, wrt[0], blp[0], wlt[1])
    s1 = _sc_segsum(DPS[1], False)(z1, src, dst, zc[DPS[1]])
    h2, z2 = _tc_combine(s1, cnt, h1, wrt[1], blp[1], wlt[2])
    s2 = _sc_segsum(DPS[2], False)(z2, src, dst, zc[DPS[2]])
    h3, z3 = _tc_combine(s2, cnt, h2, wrt[2], blp[2], wlt[3])
    s3 = _sc_segsum(DPS[3], False)(z3, src, dst, zc[DPS[3]])
    out16 = _tc_final(s3, cnt, h3, wrt[3], blp[3])
    return out16[:, :1]
